# baseline (speedup 1.0000x reference)
import functools

import jax
import jax.numpy as jnp
from jax import lax
from jax.experimental import pallas as pl
from jax.experimental.pallas import tpu as pltpu
from jax.experimental.pallas import tpu_sc as plsc

BATCH = 4096
NUM_SLOTS = 520
VOCAB = 1000000
EMB = 16

NUM_WORKERS = 32
ROWS_PER_WORKER = BATCH // NUM_WORKERS
CHUNK_ROWS = 4
CHUNK_SLOTS = CHUNK_ROWS * NUM_SLOTS
PAIR_SLOTS = 2 * NUM_SLOTS
SUPERCHUNKS = ROWS_PER_WORKER // 16
HI_MASK = -65536

PACK_WORKERS = 25
PACK_ROWS_PER_WORKER = VOCAB // PACK_WORKERS
PACK_TILE = 2000


def _pack_body(w_hbm, vtab_hbm, comb_hbm, w_v, vr_v, out_v):
    num_cores = 2
    wid = lax.axis_index("s") * num_cores + lax.axis_index("c")
    lane_iota = lax.iota(jnp.int32, 16)
    zeros16 = jnp.zeros((16,), jnp.int32)

    @pl.when(wid < PACK_WORKERS)
    def _():
        rbase = wid * PACK_ROWS_PER_WORKER

        def tile_body(t, _):
            base = rbase + t * PACK_TILE
            pltpu.sync_copy(w_hbm.at[pl.ds(base, PACK_TILE)], w_v)
            pltpu.sync_copy(vtab_hbm.at[pl.ds(base, PACK_TILE)], vr_v)

            def row_body(r, _):
                u = plsc.bitcast(vr_v[r, :], jnp.int32)
                rounded = u + 0x7FFF + ((u >> 16) & 1)
                out_v[r, :] = lax.shift_right_logical(rounded, 16)
                return 0
            lax.fori_loop(0, PACK_TILE, row_body, 0)

            def grp_body(g, _):
                wvec = w_v[pl.ds(g * 16, 16)]
                wu = plsc.bitcast(wvec, jnp.int32)
                wr = (wu + 0x7FFF + ((wu >> 16) & 1)) & HI_MASK
                rows = g * 16 + lane_iota
                cur = plsc.load_gather(out_v, [rows, zeros16])
                plsc.store_scatter(out_v, [rows, zeros16], cur | wr)
                return 0
            lax.fori_loop(0, PACK_TILE // 16, grp_body, 0)

            pltpu.sync_copy(out_v, comb_hbm.at[pl.ds(base, PACK_TILE)])
            return 0

        lax.fori_loop(0, PACK_ROWS_PER_WORKER // PACK_TILE, tile_body, 0)


def _pack_tables(w_flat, v_second):
    mesh = plsc.VectorSubcoreMesh(core_axis_name="c", subcore_axis_name="s")
    return pl.kernel(
        _pack_body,
        out_type=jax.ShapeDtypeStruct((VOCAB, EMB), jnp.int32),
        mesh=mesh,
        compiler_params=pltpu.CompilerParams(use_tc_tiling_on_sc=False,
                                             needs_layout_passes=False),
        scratch_types=[
            pltpu.VMEM((PACK_TILE,), jnp.float32),
            pltpu.VMEM((PACK_TILE, EMB), jnp.float32),
            pltpu.VMEM((PACK_TILE, EMB), jnp.int32),
        ],
    )(w_flat, v_second)


def _fm_body(vals_hbm, comb_hbm, idx_hbm, out_hbm,
             idx_v0, idx_v1, val_v0, val_v1,
             vrows_v0, vrows_v1, out_v,
             sem_v0, sem_v1):
    num_cores = 2
    wid = lax.axis_index("s") * num_cores + lax.axis_index("c")
    lane_iota = lax.iota(jnp.int32, 16)

    bufs = [
        (idx_v0, val_v0, vrows_v0, sem_v0),
        (idx_v1, val_v1, vrows_v1, sem_v1),
    ]

    def fire(gc, b):
        idx_b, val_b, vr_b, sv = bufs[b]
        base = wid * ROWS_PER_WORKER * NUM_SLOTS + gc * CHUNK_SLOTS
        pltpu.sync_copy(idx_hbm.at[pl.ds(base, CHUNK_SLOTS)], idx_b)
        pltpu.sync_copy(vals_hbm.at[pl.ds(base, CHUNK_SLOTS)], val_b)
        pltpu.async_copy(comb_hbm.at[idx_b], vr_b, sv)

    def drain(b):
        _, _, vr_b, sv = bufs[b]
        pltpu.make_async_copy(
            comb_hbm.at[pl.ds(0, CHUNK_SLOTS)], vr_b, sv).wait()

    def lane_sum(x):
        for sh in (8, 4, 2, 1):
            perm = lane_iota ^ sh
            x = x + x.at[perm].get(mode="promise_in_bounds")
        return x

    z = jnp.zeros((16,), jnp.float32)

    def slot_update(row_u, valk, acc, sq, ex):
        va = plsc.bitcast(row_u << 16, jnp.float32)
        ea = plsc.bitcast(row_u & HI_MASK, jnp.float32)
        t = va * valk
        acc = acc + t
        sq = sq + t * t
        ex = ex + ea * valk
        return acc, sq, ex

    fire(0, 0)

    def superchunk_body(sc, _):
        outvec = jnp.zeros((16,), jnp.float32)
        for sub in range(4):
            p = sub % 2
            _, val_b, vr_b, _ = bufs[p]
            gc = sc * 4 + sub
            drain(p)
            if sub < 3:
                fire(gc + 1, 1 - p)
            else:
                @pl.when(sc < SUPERCHUNKS - 1)
                def _():
                    fire(gc + 1, 1 - p)

            def half_row(base, carry0):
                acc0, sq0, ex0 = carry0

                def group(g, carry):
                    a0, a1, q0, q1, e0, e1 = carry
                    s0 = base + g * 16
                    valvec = val_b[pl.ds(s0, 16)]
                    accs = [a0, a1]
                    sqs = [q0, q1]
                    exs = [e0, e1]
                    for k in range(16):
                        j = k % 2
                        accs[j], sqs[j], exs[j] = slot_update(
                            vr_b[s0 + k, :], valvec[k],
                            accs[j], sqs[j], exs[j])
                    return (*accs, *sqs, *exs)

                a0, a1, q0, q1, e0, e1 = lax.fori_loop(
                    0, 32, group, (acc0, z, sq0, z, ex0, z))
                return a0 + a1, q0 + q1, e0 + e1

            for pair in range(2):
                pbase = pair * PAIR_SLOTS
                accA, sqA, exA = half_row(pbase, (z, z, z))
                accB, sqB, exB = z, z, z
                sm = pbase + 512
                valvec = val_b[pl.ds(sm, 16)]
                for k in range(16):
                    if k < 8:
                        accA, sqA, exA = slot_update(
                            vr_b[sm + k, :], valvec[k], accA, sqA, exA)
                    else:
                        accB, sqB, exB = slot_update(
                            vr_b[sm + k, :], valvec[k], accB, sqB, exB)
                accB, sqB, exB = half_row(pbase + 528, (accB, sqB, exB))

                for (acc, sq, ex, lane) in (
                        (accA, sqA, exA, sub * 4 + pair * 2),
                        (accB, sqB, exB, sub * 4 + pair * 2 + 1)):
                    combined = 0.5 * (acc * acc - sq) + ex
                    total = lane_sum(combined)
                    outvec = jnp.where(lane_iota == lane, total, outvec)
        out_v[pl.ds(sc * 16, 16)] = outvec
        return 0

    lax.fori_loop(0, SUPERCHUNKS, superchunk_body, 0)
    pltpu.sync_copy(out_v, out_hbm.at[pl.ds(wid * ROWS_PER_WORKER,
                                            ROWS_PER_WORKER)])


@jax.jit
def _fm_sc(vals_flat, w_flat, v_second, idx_flat):
    comb = _pack_tables(w_flat, v_second)
    mesh = plsc.VectorSubcoreMesh(core_axis_name="c", subcore_axis_name="s")
    return pl.kernel(
        _fm_body,
        out_type=jax.ShapeDtypeStruct((BATCH,), jnp.float32),
        mesh=mesh,
        compiler_params=pltpu.CompilerParams(use_tc_tiling_on_sc=False,
                                             needs_layout_passes=False),
        scratch_types=[
            pltpu.VMEM((CHUNK_SLOTS,), jnp.int32),
            pltpu.VMEM((CHUNK_SLOTS,), jnp.int32),
            pltpu.VMEM((CHUNK_SLOTS,), jnp.float32),
            pltpu.VMEM((CHUNK_SLOTS,), jnp.float32),
            pltpu.VMEM((CHUNK_SLOTS, EMB), jnp.int32),
            pltpu.VMEM((CHUNK_SLOTS, EMB), jnp.int32),
            pltpu.VMEM((ROWS_PER_WORKER,), jnp.float32),
            pltpu.SemaphoreType.DMA,
            pltpu.SemaphoreType.DMA,
        ],
    )(vals_flat, comb, idx_flat)


def kernel(feature_values, w_first, v_second, fm_bias, feature_idx):
    idx_flat = feature_idx.astype(jnp.int32).reshape(-1)
    vals_flat = feature_values.reshape(-1)
    w_flat = w_first.reshape(-1)
    logits = _fm_sc(vals_flat, w_flat, v_second, idx_flat)
    return logits + fm_bias[0]


# speedup vs baseline: 1.2925x; 1.2925x over previous
import functools

import jax
import jax.numpy as jnp
from jax import lax
from jax.experimental import pallas as pl
from jax.experimental.pallas import tpu as pltpu
from jax.experimental.pallas import tpu_sc as plsc

BATCH = 4096
NUM_SLOTS = 520
VOCAB = 1000000
EMB = 16

NUM_WORKERS = 32
ROWS_PER_WORKER = BATCH // NUM_WORKERS
CHUNK_ROWS = 4
CHUNK_SLOTS = CHUNK_ROWS * NUM_SLOTS
NUM_CHUNKS = ROWS_PER_WORKER // CHUNK_ROWS
PAIR_SLOTS = 2 * NUM_SLOTS
SUPERCHUNKS = ROWS_PER_WORKER // 16


def _fm_body(vals_hbm, w_hbm, vtab_hbm, idx_hbm, out_hbm,
             idx_v0, idx_v1, val_v0, val_v1, w_v0, w_v1,
             vrows_v0, vrows_v1, out_v,
             sem_v0, sem_v1, sem_w0, sem_w1):
    num_cores = 2
    wid = lax.axis_index("s") * num_cores + lax.axis_index("c")
    lane_iota = lax.iota(jnp.int32, 16)

    bufs = [
        (idx_v0, val_v0, w_v0, vrows_v0, sem_v0, sem_w0),
        (idx_v1, val_v1, w_v1, vrows_v1, sem_v1, sem_w1),
    ]

    def fire(gc, b):
        idx_b, val_b, w_b, vr_b, sv, sw = bufs[b]
        base = wid * ROWS_PER_WORKER * NUM_SLOTS + gc * CHUNK_SLOTS
        pltpu.sync_copy(idx_hbm.at[pl.ds(base, CHUNK_SLOTS)], idx_b)
        pltpu.sync_copy(vals_hbm.at[pl.ds(base, CHUNK_SLOTS)], val_b)
        pltpu.async_copy(vtab_hbm.at[idx_b], vr_b, sv)
        pltpu.async_copy(w_hbm.at[idx_b], w_b, sw)

    def drain(b):
        _, _, w_b, vr_b, sv, sw = bufs[b]
        pltpu.make_async_copy(
            vtab_hbm.at[pl.ds(0, CHUNK_SLOTS)], vr_b, sv).wait()
        pltpu.make_async_copy(
            w_hbm.at[pl.ds(0, CHUNK_SLOTS)], w_b, sw).wait()

    def lane_sum(x):
        for sh in (8, 4, 2, 1):
            perm = lane_iota ^ sh
            x = x + x.at[perm].get(mode="promise_in_bounds")
        return x

    z = jnp.zeros((16,), jnp.float32)

    fire(0, 0)

    def superchunk_body(sc, _):
        outvec = jnp.zeros((16,), jnp.float32)
        for sub in range(4):
            p = sub % 2
            _, val_b, w_b, vr_b, _, _ = bufs[p]
            gc = sc * 4 + sub
            drain(p)
            if sub < 3:
                fire(gc + 1, 1 - p)
            else:
                @pl.when(sc < SUPERCHUNKS - 1)
                def _():
                    fire(gc + 1, 1 - p)

            def half_row(base, carry0):
                acc0, sq0, fv0 = carry0

                def group(g, carry):
                    a0, a1, a2, a3, q0, q1, q2, q3, fv = carry
                    s0 = base + g * 16
                    valvec = val_b[pl.ds(s0, 16)]
                    wvec = w_b[pl.ds(s0, 16)]
                    fv = fv + wvec * valvec
                    accs = [a0, a1, a2, a3]
                    sqs = [q0, q1, q2, q3]
                    for k in range(16):
                        row = vr_b[s0 + k, :]
                        t = row * valvec[k]
                        accs[k % 4] = accs[k % 4] + t
                        sqs[k % 4] = sqs[k % 4] + t * t
                    return (*accs, *sqs, fv)

                a0, a1, a2, a3, q0, q1, q2, q3, fv = lax.fori_loop(
                    0, 32, group, (acc0, z, z, z, sq0, z, z, z, fv0))
                return (a0 + a1) + (a2 + a3), (q0 + q1) + (q2 + q3), fv

            for pair in range(2):
                pbase = pair * PAIR_SLOTS
                accA, sqA, fvA = half_row(pbase, (z, z, z))
                accB, sqB, fvB = z, z, z
                sm = pbase + 512
                valvec = val_b[pl.ds(sm, 16)]
                wvec = w_b[pl.ds(sm, 16)]
                wv = wvec * valvec
                fvA = fvA + jnp.where(lane_iota < 8, wv, 0.0)
                fvB = fvB + jnp.where(lane_iota < 8, 0.0, wv)
                for k in range(16):
                    row = vr_b[sm + k, :]
                    t = row * valvec[k]
                    if k < 8:
                        accA = accA + t
                        sqA = sqA + t * t
                    else:
                        accB = accB + t
                        sqB = sqB + t * t
                accB, sqB, fvB = half_row(pbase + 528, (accB, sqB, fvB))

                for (acc, sq, fv, lane) in (
                        (accA, sqA, fvA, sub * 4 + pair * 2),
                        (accB, sqB, fvB, sub * 4 + pair * 2 + 1)):
                    combined = 0.5 * (acc * acc - sq) + fv
                    total = lane_sum(combined)
                    outvec = jnp.where(lane_iota == lane, total, outvec)
        out_v[pl.ds(sc * 16, 16)] = outvec
        return 0

    lax.fori_loop(0, SUPERCHUNKS, superchunk_body, 0)
    pltpu.sync_copy(out_v, out_hbm.at[pl.ds(wid * ROWS_PER_WORKER,
                                            ROWS_PER_WORKER)])


@jax.jit
def _fm_sc(vals_flat, w_flat, v_second, idx_flat):
    mesh = plsc.VectorSubcoreMesh(core_axis_name="c", subcore_axis_name="s")
    return pl.kernel(
        _fm_body,
        out_type=jax.ShapeDtypeStruct((BATCH,), jnp.float32),
        mesh=mesh,
        compiler_params=pltpu.CompilerParams(use_tc_tiling_on_sc=False),
        scratch_types=[
            pltpu.VMEM((CHUNK_SLOTS,), jnp.int32),
            pltpu.VMEM((CHUNK_SLOTS,), jnp.int32),
            pltpu.VMEM((CHUNK_SLOTS,), jnp.float32),
            pltpu.VMEM((CHUNK_SLOTS,), jnp.float32),
            pltpu.VMEM((CHUNK_SLOTS,), jnp.float32),
            pltpu.VMEM((CHUNK_SLOTS,), jnp.float32),
            pltpu.VMEM((CHUNK_SLOTS, EMB), jnp.float32),
            pltpu.VMEM((CHUNK_SLOTS, EMB), jnp.float32),
            pltpu.VMEM((ROWS_PER_WORKER,), jnp.float32),
            pltpu.SemaphoreType.DMA,
            pltpu.SemaphoreType.DMA,
            pltpu.SemaphoreType.DMA,
            pltpu.SemaphoreType.DMA,
        ],
    )(vals_flat, w_flat, v_second, idx_flat)


def kernel(feature_values, w_first, v_second, fm_bias, feature_idx):
    idx_flat = feature_idx.astype(jnp.int32).reshape(-1)
    vals_flat = feature_values.reshape(-1)
    w_flat = w_first.reshape(-1)
    logits = _fm_sc(vals_flat, w_flat, v_second, idx_flat)
    return logits + fm_bias[0]
